# baseline (speedup 1.0000x reference)
import functools

import jax
import jax.numpy as jnp
import numpy as np
from jax import lax
from jax.experimental import pallas as pl
from jax.experimental.pallas import tpu as pltpu
from jax.experimental.pallas import tpu_sc as plsc

B = 16384
V = 1000000
D = 64
NEG = 5

NC = 2
NS = 16
L = 16
NW = NC * NS
C = B // NW
S = 128
NSUB = C // S
NT = D // L

TW = 512
VOFF = 500224
TG = VOFF // TW


def _tp_body(a_ref, b_ref, o_ref):
    o_ref[:, 0:D] = a_ref[...].T
    o_ref[:, D:2 * D] = b_ref[...].T


_tp_call = pl.pallas_call(
    _tp_body,
    grid=(TG,),
    in_specs=[pl.BlockSpec((D, TW), lambda i: (0, i)),
              pl.BlockSpec((D, TW), lambda i: (0, i + TG))],
    out_specs=pl.BlockSpec((TW, 2 * D), lambda i: (i, 0)),
    out_shape=jax.ShapeDtypeStruct((VOFF, 2 * D), jnp.float32),
)


def _sc_body(pu_h, pv_h, nv_h, emb_h, out_h, idxu, idxv, idxn, su, sv, sn,
             ru, rv, rn, dots, sem):
    wid = lax.axis_index("s") * NC + lax.axis_index("c")
    pltpu.sync_copy(pu_h.at[wid], idxu.at[pl.ds(0, C)])
    pltpu.sync_copy(pv_h.at[wid], idxv.at[pl.ds(0, C)])
    pltpu.sync_copy(nv_h.at[wid], idxn.at[pl.ds(0, C * NEG)])

    def _rows(c):
        return jnp.where(c >= VOFF, c - VOFF, c)

    for i in range(C // L):
        su[pl.ds(i * L, L)] = _rows(idxu[pl.ds(i * L, L)])
        sv[pl.ds(i * L, L)] = _rows(idxv[pl.ds(i * L, L)])
    for i in range(C * NEG // L):
        sn[pl.ds(i * L, L)] = _rows(idxn[pl.ds(i * L, L)])

    for j in range(NSUB):
        cps = [
            pltpu.async_copy(emb_h.at[su.at[pl.ds(j * S, S)]], ru, sem),
            pltpu.async_copy(emb_h.at[sv.at[pl.ds(j * S, S)]], rv, sem),
        ]
        for m in range(NEG):
            cps.append(
                pltpu.async_copy(emb_h.at[sn.at[pl.ds((j * NEG + m) * S, S)]],
                                 rn.at[pl.ds(m * S, S)], sem))
        for c in cps:
            c.wait()

        lane = lax.broadcasted_iota(jnp.int32, (L,), 0)

        def body(b, carry, j=j):
            hu = (idxu[pl.ds(j * S + b, L)][0] >= VOFF).astype(jnp.int32) * D
            hv = (idxv[pl.ds(j * S + b, L)][0] >= VOFF).astype(jnp.int32) * D
            u = [ru[b, pl.ds(hu + L * t, L)] for t in range(NT)]
            v = [rv[b, pl.ds(hv + L * t, L)] for t in range(NT)]
            mask = lane == (b & (L - 1))
            out = []
            acc = (u[0] * v[0] + u[1] * v[1]) + (u[2] * v[2] + u[3] * v[3])
            out.append(jnp.where(mask, jnp.sum(acc), carry[0]))
            for k in range(NEG):
                hn = (idxn[pl.ds((j * NEG) * S + b * NEG + k, L)][0]
                      >= VOFF).astype(jnp.int32) * D
                w = [rn[b * NEG + k, pl.ds(hn + L * t, L)] for t in range(NT)]
                acc = (u[0] * w[0] + u[1] * w[1]) + (u[2] * w[2] + u[3] * w[3])
                out.append(jnp.where(mask, jnp.sum(acc), carry[1 + k]))

            @pl.when((b & (L - 1)) == (L - 1))
            def _():
                for jd in range(6):
                    dots[jd, pl.ds(j * S + b - (L - 1), L)] = out[jd]

            return tuple(out)

        zero = jnp.zeros((L,), jnp.float32)
        lax.fori_loop(0, S, body, (zero,) * 6)

    pltpu.sync_copy(dots, out_h.at[wid])


@functools.cache
def _make_sc_call():
    return functools.partial(
        pl.kernel,
        out_type=jax.ShapeDtypeStruct((NW, 6, C), jnp.float32),
        mesh=plsc.VectorSubcoreMesh(core_axis_name="c", subcore_axis_name="s"),
        compiler_params=pltpu.CompilerParams(needs_layout_passes=False),
        scratch_types=[
            pltpu.VMEM((C + L,), jnp.int32),
            pltpu.VMEM((C + L,), jnp.int32),
            pltpu.VMEM((C * NEG + L,), jnp.int32),
            pltpu.VMEM((C,), jnp.int32),
            pltpu.VMEM((C,), jnp.int32),
            pltpu.VMEM((C * NEG,), jnp.int32),
            pltpu.VMEM((S, 2 * D), jnp.float32),
            pltpu.VMEM((S, 2 * D), jnp.float32),
            pltpu.VMEM((S * NEG, 2 * D), jnp.float32),
            pltpu.VMEM((6, C), jnp.float32),
            pltpu.SemaphoreType.DMA,
        ],
    )(_sc_body)


def _tc_body(d_ref, o_ref):
    x = d_ref[...]
    x = jnp.clip(x, -10.0, 10.0)
    sgn = jnp.where(lax.broadcasted_iota(jnp.int32, (1, 6, 1), 1) == 0,
                    -1.0, 1.0).astype(jnp.float32)
    loss = jnp.log1p(jnp.exp(x * sgn))
    o_ref[0, 0] = jnp.sum(loss) / np.float32(B)


_tc_call = pl.pallas_call(
    _tc_body,
    out_shape=jax.ShapeDtypeStruct((1, 1), jnp.float32),
    out_specs=pl.BlockSpec(memory_space=pltpu.SMEM),
)


def kernel(pos_u, pos_v, neg_v, embeddings):
    pu = pos_u.astype(jnp.int32).reshape(NW, C)
    pv = pos_v.astype(jnp.int32).reshape(NW, C)
    nv = neg_v.astype(jnp.int32).reshape(NW, C * NEG)
    embt = embeddings.T
    emb2 = _tp_call(embt, embt)
    dots = _make_sc_call()(pu, pv, nv, emb2)
    return _tc_call(dots)[0, 0]


# speedup vs baseline: 1.4388x; 1.4388x over previous
import functools

import jax
import jax.numpy as jnp
import numpy as np
from jax import lax
from jax.experimental import pallas as pl
from jax.experimental.pallas import tpu as pltpu
from jax.experimental.pallas import tpu_sc as plsc

B = 16384
V = 1000000
D = 64
NEG = 5

NC = 2
NS = 16
L = 16
NW = NC * NS
C = B // NW
S = 128
NSUB = C // S
NT = D // L

TW = 2048
TG = -(-V // TW)


def _tp_body(a_ref, o_ref):
    o_ref[:, 0:D] = a_ref[...].T


_tp_call = pl.pallas_call(
    _tp_body,
    grid=(TG,),
    in_specs=[pl.BlockSpec((D, TW), lambda i: (0, i))],
    out_specs=pl.BlockSpec((TW, 2 * D), lambda i: (i, 0)),
    out_shape=jax.ShapeDtypeStruct((V, 2 * D), jnp.float32),
)


def _sc_body(pu_h, pv_h, nv_h, emb_h, out_h, idxu, idxv, idxn, ru, rv, rn,
             dots, sem):
    wid = lax.axis_index("s") * NC + lax.axis_index("c")
    pltpu.sync_copy(pu_h.at[wid], idxu)
    pltpu.sync_copy(pv_h.at[wid], idxv)
    pltpu.sync_copy(nv_h.at[wid], idxn)

    for j in range(NSUB):
        cps = [
            pltpu.async_copy(emb_h.at[idxu.at[j]], ru, sem),
            pltpu.async_copy(emb_h.at[idxv.at[j]], rv, sem),
        ]
        for m in range(NEG):
            cps.append(
                pltpu.async_copy(emb_h.at[idxn.at[j * NEG + m]],
                                 rn.at[pl.ds(m * S, S)], sem))
        for c in cps:
            c.wait()

        lane = lax.broadcasted_iota(jnp.int32, (L,), 0)

        def body(b, carry, j=j):
            u = [ru[b, pl.ds(L * t, L)] for t in range(NT)]
            v = [rv[b, pl.ds(L * t, L)] for t in range(NT)]
            mask = lane == (b & (L - 1))
            out = []
            acc = (u[0] * v[0] + u[1] * v[1]) + (u[2] * v[2] + u[3] * v[3])
            out.append(jnp.where(mask, jnp.sum(acc), carry[0]))
            for k in range(NEG):
                w = [rn[b * NEG + k, pl.ds(L * t, L)] for t in range(NT)]
                acc = (u[0] * w[0] + u[1] * w[1]) + (u[2] * w[2] + u[3] * w[3])
                out.append(jnp.where(mask, jnp.sum(acc), carry[1 + k]))

            @pl.when((b & (L - 1)) == (L - 1))
            def _():
                for jd in range(6):
                    dots[jd, pl.ds(j * S + b - (L - 1), L)] = out[jd]

            return tuple(out)

        zero = jnp.zeros((L,), jnp.float32)
        lax.fori_loop(0, S, body, (zero,) * 6)

    pltpu.sync_copy(dots, out_h.at[wid])


@functools.cache
def _make_sc_call():
    return functools.partial(
        pl.kernel,
        out_type=jax.ShapeDtypeStruct((NW, 6, C), jnp.float32),
        mesh=plsc.VectorSubcoreMesh(core_axis_name="c", subcore_axis_name="s"),
        compiler_params=pltpu.CompilerParams(needs_layout_passes=False),
        scratch_types=[
            pltpu.VMEM((NSUB, S), jnp.int32),
            pltpu.VMEM((NSUB, S), jnp.int32),
            pltpu.VMEM((NSUB * NEG, S), jnp.int32),
            pltpu.VMEM((S, 2 * D), jnp.float32),
            pltpu.VMEM((S, 2 * D), jnp.float32),
            pltpu.VMEM((S * NEG, 2 * D), jnp.float32),
            pltpu.VMEM((6, C), jnp.float32),
            pltpu.SemaphoreType.DMA,
        ],
    )(_sc_body)


def _tc_body(d_ref, o_ref):
    x = d_ref[...]
    x = jnp.clip(x, -10.0, 10.0)
    sgn = jnp.where(lax.broadcasted_iota(jnp.int32, (1, 6, 1), 1) == 0,
                    -1.0, 1.0).astype(jnp.float32)
    loss = jnp.log1p(jnp.exp(x * sgn))
    o_ref[0, 0] = jnp.sum(loss) / np.float32(B)


_tc_call = pl.pallas_call(
    _tc_body,
    out_shape=jax.ShapeDtypeStruct((1, 1), jnp.float32),
    out_specs=pl.BlockSpec(memory_space=pltpu.SMEM),
)


def kernel(pos_u, pos_v, neg_v, embeddings):
    pu = pos_u.astype(jnp.int32).reshape(NW, NSUB, S)
    pv = pos_v.astype(jnp.int32).reshape(NW, NSUB, S)
    nv = neg_v.astype(jnp.int32).reshape(NW, NSUB * NEG, S)
    emb2 = _tp_call(embeddings.T)
    dots = _make_sc_call()(pu, pv, nv, emb2)
    return _tc_call(dots)[0, 0]
